# baseline (speedup 1.0000x reference)
import functools
import jax
import jax.numpy as jnp
from jax import lax
from jax.experimental import pallas as pl
from jax.experimental.pallas import tpu as pltpu
from jax.experimental.pallas import tpu_sc as plsc

_NUM = 1000000
_DIM = 64
_BATCH = 16384
_FIELDS = 26
_TOTAL = _BATCH * _FIELDS
_NC = 2
_NS = 16
_NW = _NC * _NS
_PER_W = _TOTAL // _NW
_CHUNK = 128
_NSTEP = _PER_W // _CHUNK
_NBUF = 4
_NITER = _NSTEP // _NBUF

_mesh = plsc.VectorSubcoreMesh(core_axis_name="c", subcore_axis_name="s")


@functools.partial(
    pl.kernel,
    out_type=jax.ShapeDtypeStruct((_TOTAL, _DIM), jnp.float32),
    mesh=_mesh,
    scratch_types=[
        pltpu.VMEM((_NSTEP, _CHUNK), jnp.int32),
        pltpu.VMEM((_NBUF, _CHUNK, _DIM), jnp.float32),
    ]
    + [pltpu.SemaphoreType.DMA] * (2 * _NBUF),
    compiler_params=pltpu.CompilerParams(use_tc_tiling_on_sc=False),
)
def _sc_gather(idx_hbm, table_hbm, out_hbm, idx_v, rows_v, *sems):
    gsem = sems[:_NBUF]
    osem = sems[_NBUF:]
    wid = lax.axis_index("s") * _NC + lax.axis_index("c")
    base = wid * _PER_W

    pltpu.sync_copy(idx_hbm.at[pl.ds(wid * _NSTEP, _NSTEP)], idx_v)

    def gather(step, b):
        pltpu.async_copy(table_hbm.at[idx_v.at[step]], rows_v.at[b], gsem[b])

    def wait_gather(b):
        pltpu.make_async_copy(table_hbm.at[idx_v.at[0]], rows_v.at[b],
                              gsem[b]).wait()

    def scatter(step, b):
        pltpu.async_copy(rows_v.at[b],
                         out_hbm.at[pl.ds(base + step * _CHUNK, _CHUNK)],
                         osem[b])

    def wait_scatter(b):
        pltpu.make_async_copy(rows_v.at[b],
                              out_hbm.at[pl.ds(base, _CHUNK)], osem[b]).wait()

    for b in range(_NBUF):
        gather(b, b)

    @pl.loop(0, _NITER)
    def _body(i):
        s0 = i * _NBUF
        for b in range(_NBUF):
            wait_gather(b)
            scatter(s0 + b, b)
        for b in range(_NBUF):
            @pl.when(s0 + b + _NBUF < _NSTEP)
            def _():
                wait_scatter(b)
                gather(s0 + b + _NBUF, b)

    for b in range(_NBUF):
        wait_scatter(b)


def kernel(idx, emb_weight):
    flat_idx = idx.astype(jnp.int32).reshape(_TOTAL // _CHUNK, _CHUNK)
    out = _sc_gather(flat_idx, emb_weight)
    return out.reshape(_BATCH, _FIELDS, _DIM)


# speedup vs baseline: 1.2286x; 1.2286x over previous
import functools
import jax
import jax.numpy as jnp
from jax import lax
from jax.experimental import pallas as pl
from jax.experimental.pallas import tpu as pltpu
from jax.experimental.pallas import tpu_sc as plsc

_NUM = 1000000
_DIM = 64
_BATCH = 16384
_FIELDS = 26
_TOTAL = _BATCH * _FIELDS
_NC = 2
_NS = 16
_NW = _NC * _NS
_B_PER_W = _BATCH // _NW
_BCHUNK = 4
_CHUNK = _BCHUNK * _FIELDS
_NSTEP = _B_PER_W // _BCHUNK
_NBUF = 4
_NITER = _NSTEP // _NBUF
_FPAD = 32
_DPAD = 128

_mesh = plsc.VectorSubcoreMesh(core_axis_name="c", subcore_axis_name="s")


@functools.partial(
    pl.kernel,
    out_type=jax.ShapeDtypeStruct((_BATCH, _FPAD, _DPAD), jnp.float32),
    mesh=_mesh,
    scratch_types=[
        pltpu.VMEM((_NSTEP, _CHUNK), jnp.int32),
        pltpu.VMEM((_NBUF, _CHUNK, _DIM), jnp.float32),
    ]
    + [pltpu.SemaphoreType.DMA] * (2 * _NBUF),
    compiler_params=pltpu.CompilerParams(use_tc_tiling_on_sc=False),
)
def _sc_gather(idx_hbm, table_hbm, out_hbm, idx_v, rows_v, *sems):
    gsem = sems[:_NBUF]
    osem = sems[_NBUF:]
    wid = lax.axis_index("s") * _NC + lax.axis_index("c")
    b_base = wid * _B_PER_W

    pltpu.sync_copy(idx_hbm.at[pl.ds(wid * _NSTEP, _NSTEP)], idx_v)

    def gather(step, b):
        pltpu.async_copy(table_hbm.at[idx_v.at[step]], rows_v.at[b], gsem[b])

    def wait_gather(b):
        pltpu.make_async_copy(table_hbm.at[idx_v.at[0]], rows_v.at[b],
                              gsem[b]).wait()

    def scatter(step, b):
        for k in range(_BCHUNK):
            src = rows_v.at[b, pl.ds(k * _FIELDS, _FIELDS)]
            dst = out_hbm.at[b_base + step * _BCHUNK + k, pl.ds(0, _FIELDS),
                             pl.ds(0, _DIM)]
            pltpu.async_copy(src, dst, osem[b])

    def wait_scatter(b):
        for k in range(_BCHUNK):
            src = rows_v.at[b, pl.ds(k * _FIELDS, _FIELDS)]
            dst = out_hbm.at[b_base + k, pl.ds(0, _FIELDS), pl.ds(0, _DIM)]
            pltpu.make_async_copy(src, dst, osem[b]).wait()

    for b in range(_NBUF):
        gather(b, b)

    @pl.loop(0, _NITER)
    def _body(i):
        s0 = i * _NBUF
        for b in range(_NBUF):
            wait_gather(b)
            scatter(s0 + b, b)
        for b in range(_NBUF):
            @pl.when(s0 + b + _NBUF < _NSTEP)
            def _():
                wait_scatter(b)
                gather(s0 + b + _NBUF, b)

    for b in range(_NBUF):
        wait_scatter(b)


def kernel(idx, emb_weight):
    flat_idx = idx.astype(jnp.int32).reshape(_BATCH // _BCHUNK, _CHUNK)
    out = _sc_gather(flat_idx, emb_weight)
    return out[:, :_FIELDS, :_DIM]


# speedup vs baseline: 1.2307x; 1.0017x over previous
import functools
import jax
import jax.numpy as jnp
from jax import lax
from jax.experimental import pallas as pl
from jax.experimental.pallas import tpu as pltpu
from jax.experimental.pallas import tpu_sc as plsc

_NUM = 1000000
_DIM = 64
_BATCH = 16384
_FIELDS = 26
_TOTAL = _BATCH * _FIELDS
_NC = 2
_NS = 16
_NW = _NC * _NS
_B_PER_W = _BATCH // _NW
_BCHUNK = 4
_CHUNK = _BCHUNK * _FIELDS
_NSTEP = _B_PER_W // _BCHUNK
_NBUF = 8
_NITER = _NSTEP // _NBUF
_FPAD = 32
_DPAD = 128

_mesh = plsc.VectorSubcoreMesh(core_axis_name="c", subcore_axis_name="s")


@functools.partial(
    pl.kernel,
    out_type=jax.ShapeDtypeStruct((_BATCH, _FPAD, _DPAD), jnp.float32),
    mesh=_mesh,
    scratch_types=[
        pltpu.VMEM((_NSTEP, _CHUNK), jnp.int32),
        pltpu.VMEM((_NBUF, _CHUNK, _DIM), jnp.float32),
    ]
    + [pltpu.SemaphoreType.DMA] * (2 * _NBUF),
    compiler_params=pltpu.CompilerParams(use_tc_tiling_on_sc=False),
)
def _sc_gather(idx_hbm, table_hbm, out_hbm, idx_v, rows_v, *sems):
    gsem = sems[:_NBUF]
    osem = sems[_NBUF:]
    wid = lax.axis_index("s") * _NC + lax.axis_index("c")
    b_base = wid * _B_PER_W

    pltpu.sync_copy(idx_hbm.at[pl.ds(wid * _NSTEP, _NSTEP)], idx_v)

    def gather(step, b):
        pltpu.async_copy(table_hbm.at[idx_v.at[step]], rows_v.at[b], gsem[b])

    def wait_gather(b):
        pltpu.make_async_copy(table_hbm.at[idx_v.at[0]], rows_v.at[b],
                              gsem[b]).wait()

    def scatter(step, b):
        for k in range(_BCHUNK):
            src = rows_v.at[b, pl.ds(k * _FIELDS, _FIELDS)]
            dst = out_hbm.at[b_base + step * _BCHUNK + k, pl.ds(0, _FIELDS),
                             pl.ds(0, _DIM)]
            pltpu.async_copy(src, dst, osem[b])

    def wait_scatter(b):
        for k in range(_BCHUNK):
            src = rows_v.at[b, pl.ds(k * _FIELDS, _FIELDS)]
            dst = out_hbm.at[b_base + k, pl.ds(0, _FIELDS), pl.ds(0, _DIM)]
            pltpu.make_async_copy(src, dst, osem[b]).wait()

    for b in range(_NBUF):
        gather(b, b)

    @pl.loop(0, _NITER)
    def _body(i):
        s0 = i * _NBUF
        for b in range(_NBUF):
            wait_gather(b)
            scatter(s0 + b, b)
        for b in range(_NBUF):
            @pl.when(s0 + b + _NBUF < _NSTEP)
            def _():
                wait_scatter(b)
                gather(s0 + b + _NBUF, b)

    for b in range(_NBUF):
        wait_scatter(b)


def kernel(idx, emb_weight):
    flat_idx = idx.astype(jnp.int32).reshape(_BATCH // _BCHUNK, _CHUNK)
    out = _sc_gather(flat_idx, emb_weight)
    return out[:, :_FIELDS, :_DIM]
